# Initial kernel scaffold; baseline (speedup 1.0000x reference)
import functools

import jax
import jax.numpy as jnp
import numpy as np
from jax import lax
from jax.experimental import pallas as pl
from jax.experimental.pallas import tpu as pltpu
from jax.experimental.pallas import tpu_sc as plsc

N_NODES = 10000
N_EDGES = 320000
D = 128
NC = 2
NS = 16
NW = NC * NS
EPW = N_EDGES // NW
CHUNK = 80
NCHUNK = EPW // CHUNK
RPS = N_NODES // NS
CNT_W = 16
BN_SCALE = 1.0 / np.sqrt(1.0 + 1e-5)


def _seg_sum_body(with_counts, *refs):
    if with_counts:
        (h_hbm, src_hbm, dst_hbm, zrow_hbm, zcnt_hbm, ones_hbm,
         agg_out, cnt_out,
         srcv, dstv, rows0, rows1, onesv, aggsh, cntsh, g0, g1) = refs
    else:
        (h_hbm, src_hbm, dst_hbm, zrow_hbm,
         agg_out,
         srcv, dstv, rows0, rows1, aggsh, g0, g1) = refs

    c = lax.axis_index("c")
    s = lax.axis_index("s")
    wid = c * NS + s

    pltpu.sync_copy(src_hbm.at[pl.ds(wid * NCHUNK, NCHUNK)], srcv)
    pltpu.sync_copy(dst_hbm.at[pl.ds(wid * NCHUNK, NCHUNK)], dstv)
    pltpu.sync_copy(zrow_hbm, aggsh.at[pl.ds(s * RPS, RPS)])
    if with_counts:
        pltpu.sync_copy(zcnt_hbm, cntsh.at[pl.ds(s * RPS, RPS)])
        pltpu.sync_copy(ones_hbm, onesv)
    plsc.subcore_barrier()

    def start_gather(ci, buf, sem):
        pltpu.async_copy(h_hbm.at[srcv.at[ci]], buf, sem)

    def wait_gather(ci, buf, sem):
        pltpu.make_async_copy(h_hbm.at[srcv.at[ci]], buf, sem).wait()

    def scatter_add(ci, buf):
        pltpu.sync_copy(buf, aggsh.at[dstv.at[ci]], add=True)
        if with_counts:
            pltpu.sync_copy(onesv, cntsh.at[dstv.at[ci]], add=True)

    start_gather(0, rows0, g0)

    @pl.loop(0, NCHUNK - 1, step=2)
    def _(ci):
        start_gather(ci + 1, rows1, g1)
        wait_gather(ci, rows0, g0)
        scatter_add(ci, rows0)
        start_gather(ci + 2, rows0, g0)
        wait_gather(ci + 1, rows1, g1)
        scatter_add(ci + 1, rows1)

    wait_gather(NCHUNK - 1, rows0, g0)
    scatter_add(NCHUNK - 1, rows0)

    plsc.subcore_barrier()
    pltpu.sync_copy(aggsh.at[pl.ds(s * RPS, RPS)],
                    agg_out.at[c].at[pl.ds(s * RPS, RPS)])
    if with_counts:
        pltpu.sync_copy(cntsh.at[pl.ds(s * RPS, RPS)],
                        cnt_out.at[c].at[pl.ds(s * RPS, RPS)])


def _make_seg_sum(with_counts):
    out_type = [jax.ShapeDtypeStruct((NC, N_NODES, D), jnp.float32)]
    scratch = [
        pltpu.VMEM((NCHUNK, CHUNK), jnp.int32),
        pltpu.VMEM((NCHUNK, CHUNK), jnp.int32),
        pltpu.VMEM((CHUNK, D), jnp.float32),
        pltpu.VMEM((CHUNK, D), jnp.float32),
    ]
    if with_counts:
        out_type.append(jax.ShapeDtypeStruct((NC, N_NODES, CNT_W), jnp.float32))
        scratch.append(pltpu.VMEM((CHUNK, CNT_W), jnp.float32))
    scratch.append(pltpu.VMEM_SHARED((N_NODES, D), jnp.float32))
    if with_counts:
        scratch.append(pltpu.VMEM_SHARED((N_NODES, CNT_W), jnp.float32))
    scratch += [pltpu.SemaphoreType.DMA, pltpu.SemaphoreType.DMA]

    mesh = plsc.VectorSubcoreMesh(core_axis_name="c", subcore_axis_name="s")
    return pl.kernel(
        functools.partial(_seg_sum_body, with_counts),
        out_type=tuple(out_type) if with_counts else out_type[0],
        mesh=mesh,
        scratch_types=scratch,
        name="seg_sum_cnt" if with_counts else "seg_sum",
    )


_seg_sum_with_counts = _make_seg_sum(True)
_seg_sum = _make_seg_sum(False)


def _tc_layer_body(relu, h_ref, pa_ref, pc_ref, wr_ref, b_ref, wt_ref,
                   g_ref, be_ref, o_ref):
    cnt = pc_ref[0, :, 0:1] + pc_ref[1, :, 0:1]
    inv = 1.0 / jnp.maximum(cnt, 1.0)
    agg = (pa_ref[0] + pa_ref[1]) * inv
    out = (jnp.dot(agg, wr_ref[...], preferred_element_type=jnp.float32)
           + b_ref[...]
           + jnp.dot(h_ref[...], wt_ref[...], preferred_element_type=jnp.float32))
    if relu:
        out = jnp.maximum(out * (g_ref[...] * BN_SCALE) + be_ref[...], 0.0)
    o_ref[...] = out


def _make_tc_layer(relu, block_rows=2000):
    grid = (N_NODES // block_rows,)
    return pl.pallas_call(
        functools.partial(_tc_layer_body, relu),
        grid=grid,
        in_specs=[
            pl.BlockSpec((block_rows, D), lambda i: (i, 0)),
            pl.BlockSpec((NC, block_rows, D), lambda i: (0, i, 0)),
            pl.BlockSpec((NC, block_rows, CNT_W), lambda i: (0, i, 0)),
            pl.BlockSpec((D, D), lambda i: (0, 0)),
            pl.BlockSpec((1, D), lambda i: (0, 0)),
            pl.BlockSpec((D, D), lambda i: (0, 0)),
            pl.BlockSpec((1, D), lambda i: (0, 0)),
            pl.BlockSpec((1, D), lambda i: (0, 0)),
        ],
        out_specs=pl.BlockSpec((block_rows, D), lambda i: (i, 0)),
        out_shape=jax.ShapeDtypeStruct((N_NODES, D), jnp.float32),
        name="gconv_dense_relu" if relu else "gconv_dense",
    )


_tc_layer_relu = _make_tc_layer(True)
_tc_layer_plain = _make_tc_layer(False)


def kernel(x, edge_index, W_rel1, b_rel1, W_root1, W_rel2, b_rel2, W_root2,
           W_rel3, b_rel3, W_root3, gamma1, beta1, gamma2, beta2):
    src = edge_index[0].astype(jnp.int32).reshape(NW * NCHUNK, CHUNK)
    dst = edge_index[1].astype(jnp.int32).reshape(NW * NCHUNK, CHUNK)
    zrow = jnp.zeros((RPS, D), jnp.float32)
    zcnt = jnp.zeros((RPS, CNT_W), jnp.float32)
    ones = jnp.ones((CHUNK, CNT_W), jnp.float32)
    r2 = lambda v: v.reshape(1, D)

    agg1, cnts = _seg_sum_with_counts(x, src, dst, zrow, zcnt, ones)
    h1 = _tc_layer_relu(x, agg1, cnts, W_rel1.T, r2(b_rel1), W_root1.T,
                        r2(gamma1), r2(beta1))
    agg2 = _seg_sum(h1, src, dst, zrow)
    h2 = _tc_layer_relu(h1, agg2, cnts, W_rel2.T, r2(b_rel2), W_root2.T,
                        r2(gamma2), r2(beta2))
    agg3 = _seg_sum(h2, src, dst, zrow)
    dummy = jnp.zeros((1, D), jnp.float32)
    h3 = _tc_layer_plain(h2, agg3, cnts, W_rel3.T, r2(b_rel3), W_root3.T,
                         dummy, dummy)
    return h3


# speedup vs baseline: 3.4536x; 3.4536x over previous
import functools

import jax
import jax.numpy as jnp
import numpy as np
from jax import lax
from jax.experimental import pallas as pl
from jax.experimental.pallas import tpu as pltpu
from jax.experimental.pallas import tpu_sc as plsc

N_NODES = 10000
N_EDGES = 320000
D = 128
NC = 2
NS = 16
NW = NC * NS
CHUNK = 128
EPW = 10240
E_PAD = NW * EPW
NCHUNK = EPW // CHUNK
IBLK = 16
NBLK = NCHUNK // IBLK
NP = 10240
RPS = NP // NS
CNT_W = 128
BN_SCALE = 1.0 / np.sqrt(1.0 + 1e-5)


def _seg_sum_body(h_hbm, src_hbm, dst_hbm, zrow_hbm, agg_out,
                  srcv, dstv, rows0, rows1, aggsh, g0, g1):
    c = lax.axis_index("c")
    s = lax.axis_index("s")
    wid = c * NS + s

    pltpu.sync_copy(zrow_hbm, aggsh.at[pl.ds(s * RPS, RPS)])
    plsc.subcore_barrier()

    def start_gather(ci, buf, sem):
        pltpu.async_copy(h_hbm.at[srcv.at[ci]], buf, sem)

    def wait_gather(ci, buf, sem):
        pltpu.make_async_copy(h_hbm.at[srcv.at[ci]], buf, sem).wait()

    def scatter_add(ci, buf):
        pltpu.sync_copy(buf, aggsh.at[dstv.at[ci]], add=True)

    @pl.loop(0, NBLK)
    def _(blk):
        pltpu.sync_copy(src_hbm.at[wid].at[pl.ds(blk * IBLK, IBLK)], srcv)
        pltpu.sync_copy(dst_hbm.at[wid].at[pl.ds(blk * IBLK, IBLK)], dstv)
        start_gather(0, rows0, g0)

        @pl.loop(0, IBLK - 2, step=2)
        def _(ci):
            start_gather(ci + 1, rows1, g1)
            wait_gather(ci, rows0, g0)
            scatter_add(ci, rows0)
            start_gather(ci + 2, rows0, g0)
            wait_gather(ci + 1, rows1, g1)
            scatter_add(ci + 1, rows1)

        start_gather(IBLK - 1, rows1, g1)
        wait_gather(IBLK - 2, rows0, g0)
        scatter_add(IBLK - 2, rows0)
        wait_gather(IBLK - 1, rows1, g1)
        scatter_add(IBLK - 1, rows1)

    plsc.subcore_barrier()
    pltpu.sync_copy(aggsh.at[pl.ds(s * RPS, RPS)],
                    agg_out.at[c].at[pl.ds(s * RPS, RPS)])


_seg_sum = pl.kernel(
    _seg_sum_body,
    out_type=jax.ShapeDtypeStruct((NC, NP, D), jnp.float32),
    mesh=plsc.VectorSubcoreMesh(core_axis_name="c", subcore_axis_name="s"),
    scratch_types=[
        pltpu.VMEM((IBLK, CHUNK), jnp.int32),
        pltpu.VMEM((IBLK, CHUNK), jnp.int32),
        pltpu.VMEM((CHUNK, D), jnp.float32),
        pltpu.VMEM((CHUNK, D), jnp.float32),
        pltpu.VMEM_SHARED((NP, D), jnp.float32),
        pltpu.SemaphoreType.DMA,
        pltpu.SemaphoreType.DMA,
    ],
    name="seg_sum",
)


def _cnt_body(dst_hbm, zcnt_hbm, ones_hbm, cnt_out, dstv, onesv, cntsh):
    c = lax.axis_index("c")
    s = lax.axis_index("s")
    wid = c * NS + s

    pltpu.sync_copy(zcnt_hbm, cntsh.at[pl.ds(s * RPS, RPS)])
    pltpu.sync_copy(ones_hbm, onesv)
    plsc.subcore_barrier()

    @pl.loop(0, NBLK)
    def _(blk):
        pltpu.sync_copy(dst_hbm.at[wid].at[pl.ds(blk * IBLK, IBLK)], dstv)

        @pl.loop(0, IBLK)
        def _(ci):
            pltpu.sync_copy(onesv, cntsh.at[dstv.at[ci]], add=True)

    plsc.subcore_barrier()
    pltpu.sync_copy(cntsh.at[pl.ds(s * RPS, RPS)],
                    cnt_out.at[c].at[pl.ds(s * RPS, RPS)])


_seg_cnt = pl.kernel(
    _cnt_body,
    out_type=jax.ShapeDtypeStruct((NC, NP, CNT_W), jnp.float32),
    mesh=plsc.VectorSubcoreMesh(core_axis_name="c", subcore_axis_name="s"),
    scratch_types=[
        pltpu.VMEM((IBLK, CHUNK), jnp.int32),
        pltpu.VMEM((CHUNK, CNT_W), jnp.float32),
        pltpu.VMEM_SHARED((NP, CNT_W), jnp.float32),
    ],
    name="seg_cnt",
)


def _tc_layer_body(relu, h_ref, pa_ref, pc_ref, wr_ref, b_ref, wt_ref,
                   g_ref, be_ref, o_ref):
    cnt = pc_ref[0, :, 0:1] + pc_ref[1, :, 0:1]
    inv = 1.0 / jnp.maximum(cnt, 1.0)
    agg = (pa_ref[0] + pa_ref[1]) * inv
    out = (jnp.dot(agg, wr_ref[...], preferred_element_type=jnp.float32)
           + b_ref[...]
           + jnp.dot(h_ref[...], wt_ref[...], preferred_element_type=jnp.float32))
    if relu:
        out = jnp.maximum(out * (g_ref[...] * BN_SCALE) + be_ref[...], 0.0)
    o_ref[...] = out


def _make_tc_layer(relu, block_rows=2000):
    grid = (N_NODES // block_rows,)
    return pl.pallas_call(
        functools.partial(_tc_layer_body, relu),
        grid=grid,
        in_specs=[
            pl.BlockSpec((block_rows, D), lambda i: (i, 0)),
            pl.BlockSpec((NC, block_rows, D), lambda i: (0, i, 0)),
            pl.BlockSpec((NC, block_rows, CNT_W), lambda i: (0, i, 0)),
            pl.BlockSpec((D, D), lambda i: (0, 0)),
            pl.BlockSpec((1, D), lambda i: (0, 0)),
            pl.BlockSpec((D, D), lambda i: (0, 0)),
            pl.BlockSpec((1, D), lambda i: (0, 0)),
            pl.BlockSpec((1, D), lambda i: (0, 0)),
        ],
        out_specs=pl.BlockSpec((block_rows, D), lambda i: (i, 0)),
        out_shape=jax.ShapeDtypeStruct((N_NODES, D), jnp.float32),
        name="gconv_dense_relu" if relu else "gconv_dense",
    )


_tc_layer_relu = _make_tc_layer(True)
_tc_layer_plain = _make_tc_layer(False)


def kernel(x, edge_index, W_rel1, b_rel1, W_root1, W_rel2, b_rel2, W_root2,
           W_rel3, b_rel3, W_root3, gamma1, beta1, gamma2, beta2):
    npad = E_PAD - N_EDGES
    src = jnp.concatenate(
        [edge_index[0].astype(jnp.int32), jnp.zeros((npad,), jnp.int32)]
    ).reshape(NW, NCHUNK, CHUNK)
    dst = jnp.concatenate(
        [edge_index[1].astype(jnp.int32),
         jnp.full((npad,), N_NODES, jnp.int32)]
    ).reshape(NW, NCHUNK, CHUNK)
    zrow = jnp.zeros((RPS, D), jnp.float32)
    zcnt = jnp.zeros((RPS, CNT_W), jnp.float32)
    ones = jnp.ones((CHUNK, CNT_W), jnp.float32)
    r2 = lambda v: v.reshape(1, D)

    cnts = _seg_cnt(dst, zcnt, ones)
    agg1 = _seg_sum(x, src, dst, zrow)
    h1 = _tc_layer_relu(x, agg1, cnts, W_rel1.T, r2(b_rel1), W_root1.T,
                        r2(gamma1), r2(beta1))
    agg2 = _seg_sum(h1, src, dst, zrow)
    h2 = _tc_layer_relu(h1, agg2, cnts, W_rel2.T, r2(b_rel2), W_root2.T,
                        r2(gamma2), r2(beta2))
    agg3 = _seg_sum(h2, src, dst, zrow)
    dummy = jnp.zeros((1, D), jnp.float32)
    h3 = _tc_layer_plain(h2, agg3, cnts, W_rel3.T, r2(b_rel3), W_root3.T,
                         dummy, dummy)
    return h3
